# Initial kernel scaffold; baseline (speedup 1.0000x reference)
import jax
import jax.numpy as jnp
from jax.experimental import pallas as pl


def kernel(weight, scores):
    raise NotImplementedError("write your pallas kernel here")


# speedup vs baseline: 22.6147x; 22.6147x over previous
import functools

import jax
import jax.numpy as jnp
from jax import lax
from jax.experimental import pallas as pl
from jax.experimental.pallas import tpu as pltpu
from jax.experimental.pallas import tpu_sc as plsc

_N = 4096 * 4096
_NC = 2
_NS = 16
_NW = _NC * _NS
_L = 16
_PER_W = _N // _NW
_CHUNK = 16384
_NPAIR = _PER_W // (2 * _CHUNK)


def _make_hist_kernel(shift, nbins, maskc):
    hist_words = nbins * _L
    mesh = plsc.VectorSubcoreMesh(core_axis_name="c", subcore_axis_name="s")

    @functools.partial(
        pl.kernel,
        mesh=mesh,
        compiler_params=pltpu.CompilerParams(needs_layout_passes=False),
        out_type=jax.ShapeDtypeStruct((_NW, hist_words), jnp.int32),
        scratch_types=[
            pltpu.VMEM((_CHUNK,), jnp.float32),
            pltpu.VMEM((_CHUNK,), jnp.float32),
            pltpu.VMEM((hist_words,), jnp.int32),
            pltpu.VMEM((_L,), jnp.uint32),
            pltpu.SemaphoreType.DMA,
            pltpu.SemaphoreType.DMA,
        ],
    )
    def hist_kernel(scores_hbm, prefix_hbm, out_hbm, buf0, buf1, hist,
                    pref_v, sem0, sem1):
        wid = lax.axis_index("s") * _NC + lax.axis_index("c")
        base = wid * _PER_W

        zeros = jnp.zeros((_L,), jnp.int32)

        def zbody(i, _):
            hist[pl.ds(i * _L, _L)] = zeros
            return 0

        lax.fori_loop(0, nbins, zbody, 0)

        pltpu.sync_copy(prefix_hbm, pref_v)
        pv = pref_v[...]
        lane = lax.iota(jnp.int32, _L)
        ones = jnp.ones((_L,), jnp.int32)

        def process(buf):
            def body(j, _):
                v = buf[pl.ds(j * _L, _L)]
                bu = lax.bitcast_convert_type(v, jnp.uint32)
                sign = lax.shift_right_logical(bu, jnp.uint32(31))
                flip = (jnp.uint32(0) - sign) | jnp.uint32(0x80000000)
                key = bu ^ flip
                bucket = lax.shift_right_logical(key, jnp.uint32(shift)) \
                    & jnp.uint32(nbins - 1)
                idx = (lax.bitcast_convert_type(bucket, jnp.int32) << 4) | lane
                m = (key & jnp.uint32(maskc)) == pv
                plsc.addupdate_scatter(hist, [idx], ones, mask=m)
                return 0

            lax.fori_loop(0, _CHUNK // _L, body, 0)

        def pair(p, _):
            c0 = base + (2 * p) * _CHUNK
            cp0 = pltpu.async_copy(scores_hbm.at[pl.ds(c0, _CHUNK)], buf0, sem0)
            cp1 = pltpu.async_copy(scores_hbm.at[pl.ds(c0 + _CHUNK, _CHUNK)],
                                   buf1, sem1)
            cp0.wait()
            process(buf0)
            cp1.wait()
            process(buf1)
            return 0

        lax.fori_loop(0, _NPAIR, pair, 0)

        pltpu.sync_copy(hist, out_hbm.at[wid])

    return hist_kernel


_hist_a = _make_hist_kernel(20, 4096, 0)
_hist_b = _make_hist_kernel(8, 4096, 0xFFF00000)
_hist_c = _make_hist_kernel(0, 256, 0xFFFFFF00)


def _select(hist_flat, nbins, rank):
    h = hist_flat.reshape(_NW, nbins, _L).sum(axis=(0, 2))
    c = jnp.cumsum(h)
    b = jnp.argmax(c >= rank)
    within = rank - (c[b] - h[b])
    return b.astype(jnp.uint32), within


def _apply_body(thr_ref, w_ref, s_ref, o_ref):
    thr = thr_ref[0, 0]
    o_ref[...] = jnp.where(s_ref[...] < thr, jnp.float32(0.0), w_ref[...])


_apply = pl.pallas_call(
    _apply_body,
    grid=(16,),
    in_specs=[
        pl.BlockSpec(memory_space=pltpu.SMEM),
        pl.BlockSpec((256, 4096), lambda i: (i, 0)),
        pl.BlockSpec((256, 4096), lambda i: (i, 0)),
    ],
    out_specs=pl.BlockSpec((256, 4096), lambda i: (i, 0)),
    out_shape=jax.ShapeDtypeStruct((4096, 4096), jnp.float32),
)


def kernel(weight, scores):
    n = scores.size
    k = jnp.int32(int(1 + round(0.9 * (n - 1))))

    scores_flat = scores.reshape(-1)

    hist = _hist_a(scores_flat, jnp.zeros((_L,), jnp.uint32))
    b1, r1 = _select(hist, 4096, k)
    p1 = b1 << 20

    hist = _hist_b(scores_flat, jnp.broadcast_to(p1, (_L,)))
    b2, r2 = _select(hist, 4096, r1)
    p2 = p1 | (b2 << 8)

    hist = _hist_c(scores_flat, jnp.broadcast_to(p2, (_L,)))
    b3, _ = _select(hist, 256, r2)
    key = p2 | b3

    bits = jnp.where(key >= jnp.uint32(0x80000000),
                     key ^ jnp.uint32(0x80000000), ~key)
    thr = lax.bitcast_convert_type(bits, jnp.float32)

    return _apply(thr.reshape(1, 1), weight, scores)


# speedup vs baseline: 106.9314x; 4.7284x over previous
import functools

import jax
import jax.numpy as jnp
from jax import lax
from jax.experimental import pallas as pl
from jax.experimental.pallas import tpu as pltpu
from jax.experimental.pallas import tpu_sc as plsc

_N = 4096 * 4096
_NC = 2
_NS = 16
_NW = _NC * _NS
_L = 16
_PER_W = _N // _NW
_SLAB_R = 8
_SLAB_C = 2048
_SLAB_ELEMS = _SLAB_R * _SLAB_C
_SLABS_PER_W = _PER_W // _SLAB_ELEMS
_SLABS_PER_ROWBAND = 4096 // _SLAB_C


def _make_hist_kernel(shift, nbins, maskc):
    hist_words = nbins * _L
    mesh = plsc.VectorSubcoreMesh(core_axis_name="c", subcore_axis_name="s")

    @functools.partial(
        pl.kernel,
        mesh=mesh,
        compiler_params=pltpu.CompilerParams(needs_layout_passes=False),
        out_type=jax.ShapeDtypeStruct((_NW, hist_words), jnp.int32),
        scratch_types=[
            pltpu.VMEM((_SLAB_R, _SLAB_C), jnp.float32),
            pltpu.VMEM((_SLAB_R, _SLAB_C), jnp.float32),
            pltpu.VMEM((hist_words,), jnp.int32),
            pltpu.VMEM((_L,), jnp.uint32),
            pltpu.SemaphoreType.DMA,
            pltpu.SemaphoreType.DMA,
        ],
    )
    def hist_kernel(scores_hbm, prefix_hbm, out_hbm, buf0, buf1, hist,
                    pref_v, sem0, sem1):
        wid = lax.axis_index("s") * _NC + lax.axis_index("c")

        zeros = jnp.zeros((_L,), jnp.int32)

        @plsc.parallel_loop(0, nbins, unroll=8)
        def _zero(i):
            hist[pl.ds(i * _L, _L)] = zeros

        pltpu.sync_copy(prefix_hbm, pref_v)
        pv = pref_v[...]
        lane = lax.iota(jnp.int32, _L)
        ones = jnp.ones((_L,), jnp.int32)

        def slab_src(l):
            g = wid * _SLABS_PER_W + l
            r0 = (g // _SLABS_PER_ROWBAND) * _SLAB_R
            c0 = (g % _SLABS_PER_ROWBAND) * _SLAB_C
            return scores_hbm.at[pl.ds(r0, _SLAB_R), pl.ds(c0, _SLAB_C)]

        def process(buf):
            @plsc.parallel_loop(0, _SLAB_ELEMS // _L, unroll=8)
            def _body(j):
                r = lax.shift_right_logical(j, 7)
                c = (j & (_SLAB_C // _L - 1)) * _L
                v = buf[r, pl.ds(c, _L)]
                bu = lax.bitcast_convert_type(v, jnp.uint32)
                sign = lax.shift_right_arithmetic(
                    lax.bitcast_convert_type(v, jnp.int32), 31)
                flip = lax.bitcast_convert_type(sign, jnp.uint32) \
                    | jnp.uint32(0x80000000)
                key = bu ^ flip
                bucket = lax.shift_right_logical(key, jnp.uint32(shift)) \
                    & jnp.uint32(nbins - 1)
                idx = (lax.bitcast_convert_type(bucket, jnp.int32) << 4) \
                    | lane
                m = (key & jnp.uint32(maskc)) == pv
                plsc.addupdate_scatter(hist, [idx], ones, mask=m)

        last = _SLABS_PER_W - 1
        pltpu.make_async_copy(slab_src(0), buf0, sem0).start()
        pltpu.make_async_copy(slab_src(1), buf1, sem1).start()

        def pair(p, _):
            l0 = 2 * p
            pltpu.make_async_copy(slab_src(l0), buf0, sem0).wait()
            process(buf0)
            pltpu.make_async_copy(
                slab_src(jnp.minimum(l0 + 2, last)), buf0, sem0).start()
            pltpu.make_async_copy(slab_src(l0 + 1), buf1, sem1).wait()
            process(buf1)
            pltpu.make_async_copy(
                slab_src(jnp.minimum(l0 + 3, last)), buf1, sem1).start()
            return 0

        lax.fori_loop(0, _SLABS_PER_W // 2, pair, 0)
        pltpu.make_async_copy(slab_src(last), buf0, sem0).wait()
        pltpu.make_async_copy(slab_src(last), buf1, sem1).wait()

        pltpu.sync_copy(hist, out_hbm.at[wid])

    return hist_kernel


_hist_a = _make_hist_kernel(20, 4096, 0)
_hist_b = _make_hist_kernel(8, 4096, 0xFFF00000)
_hist_c = _make_hist_kernel(0, 256, 0xFFFFFF00)


def _select(hist_flat, nbins, rank):
    h = hist_flat.reshape(_NW, nbins, _L).sum(axis=(0, 2))
    c = jnp.cumsum(h)
    b = jnp.argmax(c >= rank)
    within = rank - (c[b] - h[b])
    return b.astype(jnp.uint32), within


def _apply_body(thr_ref, w_ref, s_ref, o_ref):
    thr = thr_ref[0, 0]
    o_ref[...] = jnp.where(s_ref[...] < thr, jnp.float32(0.0), w_ref[...])


_apply = pl.pallas_call(
    _apply_body,
    grid=(16,),
    in_specs=[
        pl.BlockSpec(memory_space=pltpu.SMEM),
        pl.BlockSpec((256, 4096), lambda i: (i, 0)),
        pl.BlockSpec((256, 4096), lambda i: (i, 0)),
    ],
    out_specs=pl.BlockSpec((256, 4096), lambda i: (i, 0)),
    out_shape=jax.ShapeDtypeStruct((4096, 4096), jnp.float32),
)


def kernel(weight, scores):
    n = scores.size
    k = jnp.int32(int(1 + round(0.9 * (n - 1))))

    hist = _hist_a(scores, jnp.zeros((_L,), jnp.uint32))
    b1, r1 = _select(hist, 4096, k)
    p1 = b1 << 20

    hist = _hist_b(scores, jnp.broadcast_to(p1, (_L,)))
    b2, r2 = _select(hist, 4096, r1)
    p2 = p1 | (b2 << 8)

    hist = _hist_c(scores, jnp.broadcast_to(p2, (_L,)))
    b3, _ = _select(hist, 256, r2)
    key = p2 | b3

    bits = jnp.where(key >= jnp.uint32(0x80000000),
                     key ^ jnp.uint32(0x80000000), ~key)
    thr = lax.bitcast_convert_type(bits, jnp.float32)

    return _apply(thr.reshape(1, 1), weight, scores)


# speedup vs baseline: 115.6641x; 1.0817x over previous
import functools

import jax
import jax.numpy as jnp
from jax import lax
from jax.experimental import pallas as pl
from jax.experimental.pallas import tpu as pltpu
from jax.experimental.pallas import tpu_sc as plsc

_N = 4096 * 4096
_NC = 2
_NS = 16
_NW = _NC * _NS
_L = 16
_PER_W = _N // _NW
_SLAB_R = 8
_SLAB_C = 2048
_SLAB_ELEMS = _SLAB_R * _SLAB_C
_SLABS_PER_W = _PER_W // _SLAB_ELEMS
_SLABS_PER_ROWBAND = 4096 // _SLAB_C


def _make_hist_kernel(shift, nbins, maskc):
    hist_words = nbins * _L
    mesh = plsc.VectorSubcoreMesh(core_axis_name="c", subcore_axis_name="s")

    @functools.partial(
        pl.kernel,
        mesh=mesh,
        compiler_params=pltpu.CompilerParams(needs_layout_passes=False),
        out_type=jax.ShapeDtypeStruct((_NW, hist_words), jnp.int32),
        scratch_types=[
            pltpu.VMEM((_SLAB_R, _SLAB_C), jnp.float32),
            pltpu.VMEM((_SLAB_R, _SLAB_C), jnp.float32),
            pltpu.VMEM((hist_words,), jnp.int32),
            pltpu.VMEM((_L,), jnp.uint32),
            pltpu.SemaphoreType.DMA,
            pltpu.SemaphoreType.DMA,
        ],
    )
    def hist_kernel(scores_hbm, prefix_hbm, out_hbm, buf0, buf1, hist,
                    pref_v, sem0, sem1):
        wid = lax.axis_index("s") * _NC + lax.axis_index("c")

        zeros = jnp.zeros((_L,), jnp.int32)

        @plsc.parallel_loop(0, nbins, unroll=8)
        def _zero(i):
            hist[pl.ds(i * _L, _L)] = zeros

        pltpu.sync_copy(prefix_hbm, pref_v)
        pv = pref_v[...]
        lane = lax.iota(jnp.int32, _L)
        ones = jnp.ones((_L,), jnp.int32)

        def slab_src(l):
            g = wid * _SLABS_PER_W + l
            r0 = (g // _SLABS_PER_ROWBAND) * _SLAB_R
            c0 = (g % _SLABS_PER_ROWBAND) * _SLAB_C
            return scores_hbm.at[pl.ds(r0, _SLAB_R), pl.ds(c0, _SLAB_C)]

        def process(buf):
            @plsc.parallel_loop(0, _SLAB_ELEMS // _L, unroll=8)
            def _body(j):
                r = lax.shift_right_logical(j, 7)
                c = (j & (_SLAB_C // _L - 1)) * _L
                v = buf[r, pl.ds(c, _L)]
                bu = lax.bitcast_convert_type(v, jnp.uint32)
                if shift >= 4:
                    t = lax.shift_right_logical(bu, jnp.uint32(shift - 4))
                else:
                    t = lax.shift_left(bu, jnp.uint32(4 - shift))
                idx = lax.bitcast_convert_type(
                    t & jnp.uint32((nbins - 1) * _L), jnp.int32) | lane
                m = (bu & jnp.uint32(maskc)) == pv
                plsc.addupdate_scatter(hist, [idx], ones, mask=m)

        last = _SLABS_PER_W - 1
        pltpu.make_async_copy(slab_src(0), buf0, sem0).start()
        pltpu.make_async_copy(slab_src(1), buf1, sem1).start()

        def pair(p, _):
            l0 = 2 * p
            pltpu.make_async_copy(slab_src(l0), buf0, sem0).wait()
            process(buf0)
            pltpu.make_async_copy(
                slab_src(jnp.minimum(l0 + 2, last)), buf0, sem0).start()
            pltpu.make_async_copy(slab_src(l0 + 1), buf1, sem1).wait()
            process(buf1)
            pltpu.make_async_copy(
                slab_src(jnp.minimum(l0 + 3, last)), buf1, sem1).start()
            return 0

        lax.fori_loop(0, _SLABS_PER_W // 2, pair, 0)
        pltpu.make_async_copy(slab_src(last), buf0, sem0).wait()
        pltpu.make_async_copy(slab_src(last), buf1, sem1).wait()

        pltpu.sync_copy(hist, out_hbm.at[wid])

    return hist_kernel


_hist_a = _make_hist_kernel(20, 4096, 0)
_hist_b = _make_hist_kernel(8, 4096, 0xFFF00000)
_hist_c = _make_hist_kernel(0, 256, 0xFFFFFF00)


def _reduce(hist_flat, nbins):
    return hist_flat.reshape(_NW, nbins, _L).sum(axis=(0, 2))


def _pick(h, rank):
    c = jnp.cumsum(h)
    b = jnp.argmax(c >= rank)
    within = rank - (c[b] - h[b])
    return b, within


def _apply_body(thr_ref, w_ref, s_ref, o_ref):
    thr = thr_ref[0, 0]
    o_ref[...] = jnp.where(s_ref[...] < thr, jnp.float32(0.0), w_ref[...])


_apply = pl.pallas_call(
    _apply_body,
    grid=(16,),
    in_specs=[
        pl.BlockSpec(memory_space=pltpu.SMEM),
        pl.BlockSpec((256, 4096), lambda i: (i, 0)),
        pl.BlockSpec((256, 4096), lambda i: (i, 0)),
    ],
    out_specs=pl.BlockSpec((256, 4096), lambda i: (i, 0)),
    out_shape=jax.ShapeDtypeStruct((4096, 4096), jnp.float32),
)


def kernel(weight, scores):
    n = scores.size
    k = jnp.int32(int(1 + round(0.9 * (n - 1))))

    h_raw = _reduce(_hist_a(scores, jnp.zeros((_L,), jnp.uint32)), 4096)
    h_key = jnp.concatenate([h_raw[2048:][::-1], h_raw[:2048]])
    b1, r1 = _pick(h_key, k)
    neg = b1 < 2048
    r1raw = jnp.where(neg, 4095 - b1, b1 - 2048)
    p1 = r1raw.astype(jnp.uint32) << 20

    h2 = _reduce(_hist_b(scores, jnp.broadcast_to(p1, (_L,))), 4096)
    b2, r2 = _pick(jnp.where(neg, h2[::-1], h2), r1)
    m2raw = jnp.where(neg, 4095 - b2, b2)
    p2 = p1 | (m2raw.astype(jnp.uint32) << 8)

    h3 = _reduce(_hist_c(scores, jnp.broadcast_to(p2, (_L,))), 256)
    b3, _ = _pick(jnp.where(neg, h3[::-1], h3), r2)
    lowraw = jnp.where(neg, 255 - b3, b3)

    bits = p2 | lowraw.astype(jnp.uint32)
    thr = lax.bitcast_convert_type(bits, jnp.float32)

    return _apply(thr.reshape(1, 1), weight, scores)


# speedup vs baseline: 142.8973x; 1.2355x over previous
import functools

import jax
import jax.numpy as jnp
from jax import lax
from jax.experimental import pallas as pl
from jax.experimental.pallas import tpu as pltpu
from jax.experimental.pallas import tpu_sc as plsc

_N = 4096 * 4096
_NC = 2
_NS = 16
_NW = _NC * _NS
_L = 16
_PER_W = _N // _NW
_SLAB_R = 8
_SLAB_C = 2048
_SLAB_ELEMS = _SLAB_R * _SLAB_C
_SLABS_PER_W = _PER_W // _SLAB_ELEMS
_SLABS_PER_ROWBAND = 4096 // _SLAB_C


def _make_hist_kernel(shift, nbins, maskc):
    hist_words = nbins * _L
    mesh = plsc.VectorSubcoreMesh(core_axis_name="c", subcore_axis_name="s")

    @functools.partial(
        pl.kernel,
        mesh=mesh,
        compiler_params=pltpu.CompilerParams(needs_layout_passes=False),
        out_type=jax.ShapeDtypeStruct((_NW, hist_words), jnp.int32),
        scratch_types=[
            pltpu.VMEM((_SLAB_R, _SLAB_C), jnp.float32),
            pltpu.VMEM((_SLAB_R, _SLAB_C), jnp.float32),
            pltpu.VMEM((hist_words,), jnp.int32),
            pltpu.VMEM((_L,), jnp.uint32),
            pltpu.SemaphoreType.DMA,
            pltpu.SemaphoreType.DMA,
        ],
    )
    def hist_kernel(scores_hbm, prefix_hbm, out_hbm, buf0, buf1, hist,
                    pref_v, sem0, sem1):
        wid = lax.axis_index("s") * _NC + lax.axis_index("c")

        zeros = jnp.zeros((_L,), jnp.int32)

        @plsc.parallel_loop(0, nbins, unroll=8)
        def _zero(i):
            hist[pl.ds(i * _L, _L)] = zeros

        pltpu.sync_copy(prefix_hbm, pref_v)
        pv = pref_v[...]
        lane = lax.iota(jnp.int32, _L)
        ones = jnp.ones((_L,), jnp.int32)

        def slab_src(l):
            g = wid * _SLABS_PER_W + l
            r0 = (g // _SLABS_PER_ROWBAND) * _SLAB_R
            c0 = (g % _SLABS_PER_ROWBAND) * _SLAB_C
            return scores_hbm.at[pl.ds(r0, _SLAB_R), pl.ds(c0, _SLAB_C)]

        def process(buf):
            @plsc.parallel_loop(0, _SLAB_ELEMS // _L, unroll=8)
            def _body(j):
                r = lax.shift_right_logical(j, 7)
                c = (j & (_SLAB_C // _L - 1)) * _L
                v = buf[r, pl.ds(c, _L)]
                bu = lax.bitcast_convert_type(v, jnp.uint32)
                if shift >= 4:
                    t = lax.shift_right_logical(bu, jnp.uint32(shift - 4))
                else:
                    t = lax.shift_left(bu, jnp.uint32(4 - shift))
                idx = lax.bitcast_convert_type(
                    t & jnp.uint32((nbins - 1) * _L), jnp.int32) | lane
                m = (bu & jnp.uint32(maskc)) == pv
                plsc.addupdate_scatter(hist, [idx], ones, mask=m)

        last = _SLABS_PER_W - 1
        pltpu.make_async_copy(slab_src(0), buf0, sem0).start()
        pltpu.make_async_copy(slab_src(1), buf1, sem1).start()

        def pair(p, _):
            l0 = 2 * p
            pltpu.make_async_copy(slab_src(l0), buf0, sem0).wait()
            process(buf0)
            pltpu.make_async_copy(
                slab_src(jnp.minimum(l0 + 2, last)), buf0, sem0).start()
            pltpu.make_async_copy(slab_src(l0 + 1), buf1, sem1).wait()
            process(buf1)
            pltpu.make_async_copy(
                slab_src(jnp.minimum(l0 + 3, last)), buf1, sem1).start()
            return 0

        lax.fori_loop(0, _SLABS_PER_W // 2, pair, 0)
        pltpu.make_async_copy(slab_src(last), buf0, sem0).wait()
        pltpu.make_async_copy(slab_src(last), buf1, sem1).wait()

        pltpu.sync_copy(hist, out_hbm.at[wid])

    return hist_kernel


_hist_a = _make_hist_kernel(20, 4096, 0)
_hist_b = _make_hist_kernel(8, 4096, 0xFFF00000)
_hist_c = _make_hist_kernel(0, 256, 0xFFFFFF00)


def _make_hist16_kernel(top):
    nbins = 65536
    mesh = plsc.VectorSubcoreMesh(core_axis_name="c", subcore_axis_name="s")

    @functools.partial(
        pl.kernel,
        mesh=mesh,
        compiler_params=pltpu.CompilerParams(needs_layout_passes=False),
        out_type=jax.ShapeDtypeStruct((_NW, nbins), jnp.int32),
        scratch_types=[
            pltpu.VMEM((_SLAB_R, _SLAB_C), jnp.float32),
            pltpu.VMEM((_SLAB_R, _SLAB_C), jnp.float32),
            pltpu.VMEM((nbins,), jnp.int32),
            pltpu.VMEM((_L,), jnp.uint32),
            pltpu.SemaphoreType.DMA,
            pltpu.SemaphoreType.DMA,
        ],
    )
    def hist_kernel(scores_hbm, prefix_hbm, out_hbm, buf0, buf1, hist,
                    pref_v, sem0, sem1):
        wid = lax.axis_index("s") * _NC + lax.axis_index("c")

        zeros = jnp.zeros((_L,), jnp.int32)

        @plsc.parallel_loop(0, nbins // _L, unroll=8)
        def _zero(i):
            hist[pl.ds(i * _L, _L)] = zeros

        pltpu.sync_copy(prefix_hbm, pref_v)
        pv = pref_v[...]

        def slab_src(l):
            g = wid * _SLABS_PER_W + l
            r0 = (g // _SLABS_PER_ROWBAND) * _SLAB_R
            c0 = (g % _SLABS_PER_ROWBAND) * _SLAB_C
            return scores_hbm.at[pl.ds(r0, _SLAB_R), pl.ds(c0, _SLAB_C)]

        def process(buf):
            @plsc.parallel_loop(0, _SLAB_ELEMS // _L, unroll=8)
            def _body(j):
                r = lax.shift_right_logical(j, 7)
                c = (j & (_SLAB_C // _L - 1)) * _L
                v = buf[r, pl.ds(c, _L)]
                bu = lax.bitcast_convert_type(v, jnp.uint32)
                if top:
                    bucket = lax.shift_right_logical(bu, jnp.uint32(16))
                    m = (bu & jnp.uint32(0)) == pv
                else:
                    bucket = bu & jnp.uint32(0xFFFF)
                    m = (bu & jnp.uint32(0xFFFF0000)) == pv
                idx = lax.bitcast_convert_type(bucket, jnp.int32)
                cnt, last = plsc.scan_count(idx, mask=m)
                plsc.addupdate_scatter(hist, [idx], cnt, mask=last)

        last_slab = _SLABS_PER_W - 1
        pltpu.make_async_copy(slab_src(0), buf0, sem0).start()
        pltpu.make_async_copy(slab_src(1), buf1, sem1).start()

        def pair(p, _):
            l0 = 2 * p
            pltpu.make_async_copy(slab_src(l0), buf0, sem0).wait()
            process(buf0)
            pltpu.make_async_copy(
                slab_src(jnp.minimum(l0 + 2, last_slab)), buf0, sem0).start()
            pltpu.make_async_copy(slab_src(l0 + 1), buf1, sem1).wait()
            process(buf1)
            pltpu.make_async_copy(
                slab_src(jnp.minimum(l0 + 3, last_slab)), buf1, sem1).start()
            return 0

        lax.fori_loop(0, _SLABS_PER_W // 2, pair, 0)
        pltpu.make_async_copy(slab_src(last_slab), buf0, sem0).wait()
        pltpu.make_async_copy(slab_src(last_slab), buf1, sem1).wait()

        pltpu.sync_copy(hist, out_hbm.at[wid])

    return hist_kernel


_hist16_hi = _make_hist16_kernel(True)
_hist16_lo = _make_hist16_kernel(False)


def _reduce(hist_flat, nbins):
    return hist_flat.reshape(_NW, nbins, _L).sum(axis=(0, 2))


def _pick(h, rank):
    c = jnp.cumsum(h)
    b = jnp.argmax(c >= rank)
    within = rank - (c[b] - h[b])
    return b, within


def _apply_body(thr_ref, w_ref, s_ref, o_ref):
    thr = thr_ref[0, 0]
    o_ref[...] = jnp.where(s_ref[...] < thr, jnp.float32(0.0), w_ref[...])


_apply = pl.pallas_call(
    _apply_body,
    grid=(16,),
    in_specs=[
        pl.BlockSpec(memory_space=pltpu.SMEM),
        pl.BlockSpec((256, 4096), lambda i: (i, 0)),
        pl.BlockSpec((256, 4096), lambda i: (i, 0)),
    ],
    out_specs=pl.BlockSpec((256, 4096), lambda i: (i, 0)),
    out_shape=jax.ShapeDtypeStruct((4096, 4096), jnp.float32),
)


def kernel(weight, scores):
    n = scores.size
    k = jnp.int32(int(1 + round(0.9 * (n - 1))))

    h_raw = _hist16_hi(scores, jnp.zeros((_L,), jnp.uint32)).sum(axis=0)
    h_key = jnp.concatenate([h_raw[32768:][::-1], h_raw[:32768]])
    b1, r1 = _pick(h_key, k)
    neg = b1 < 32768
    r1raw = jnp.where(neg, 65535 - b1, b1 - 32768)
    p1 = r1raw.astype(jnp.uint32) << 16

    h2 = _hist16_lo(scores, jnp.broadcast_to(p1, (_L,))).sum(axis=0)
    b2, _ = _pick(jnp.where(neg, h2[::-1], h2), r1)
    lowraw = jnp.where(neg, 65535 - b2, b2)

    bits = p1 | lowraw.astype(jnp.uint32)
    thr = lax.bitcast_convert_type(bits, jnp.float32)

    return _apply(thr.reshape(1, 1), weight, scores)


# speedup vs baseline: 146.7264x; 1.0268x over previous
import functools

import jax
import jax.numpy as jnp
from jax import lax
from jax.experimental import pallas as pl
from jax.experimental.pallas import tpu as pltpu
from jax.experimental.pallas import tpu_sc as plsc

_N = 4096 * 4096
_NC = 2
_NS = 16
_NW = _NC * _NS
_L = 16
_PER_W = _N // _NW
_SLAB_R = 8
_SLAB_C = 2048
_SLAB_ELEMS = _SLAB_R * _SLAB_C
_SLABS_PER_W = _PER_W // _SLAB_ELEMS
_SLABS_PER_ROWBAND = 4096 // _SLAB_C


def _make_hist_kernel(shift, nbins, maskc):
    hist_words = nbins * _L
    mesh = plsc.VectorSubcoreMesh(core_axis_name="c", subcore_axis_name="s")

    @functools.partial(
        pl.kernel,
        mesh=mesh,
        compiler_params=pltpu.CompilerParams(needs_layout_passes=False),
        out_type=jax.ShapeDtypeStruct((_NW, hist_words), jnp.int32),
        scratch_types=[
            pltpu.VMEM((_SLAB_R, _SLAB_C), jnp.float32),
            pltpu.VMEM((_SLAB_R, _SLAB_C), jnp.float32),
            pltpu.VMEM((hist_words,), jnp.int32),
            pltpu.VMEM((_L,), jnp.uint32),
            pltpu.SemaphoreType.DMA,
            pltpu.SemaphoreType.DMA,
        ],
    )
    def hist_kernel(scores_hbm, prefix_hbm, out_hbm, buf0, buf1, hist,
                    pref_v, sem0, sem1):
        wid = lax.axis_index("s") * _NC + lax.axis_index("c")

        zeros = jnp.zeros((_L,), jnp.int32)

        @plsc.parallel_loop(0, nbins, unroll=8)
        def _zero(i):
            hist[pl.ds(i * _L, _L)] = zeros

        pltpu.sync_copy(prefix_hbm, pref_v)
        pv = pref_v[...]
        lane = lax.iota(jnp.int32, _L)
        ones = jnp.ones((_L,), jnp.int32)

        def slab_src(l):
            g = wid * _SLABS_PER_W + l
            r0 = (g // _SLABS_PER_ROWBAND) * _SLAB_R
            c0 = (g % _SLABS_PER_ROWBAND) * _SLAB_C
            return scores_hbm.at[pl.ds(r0, _SLAB_R), pl.ds(c0, _SLAB_C)]

        def process(buf):
            @plsc.parallel_loop(0, _SLAB_ELEMS // _L, unroll=8)
            def _body(j):
                r = lax.shift_right_logical(j, 7)
                c = (j & (_SLAB_C // _L - 1)) * _L
                v = buf[r, pl.ds(c, _L)]
                bu = lax.bitcast_convert_type(v, jnp.uint32)
                if shift >= 4:
                    t = lax.shift_right_logical(bu, jnp.uint32(shift - 4))
                else:
                    t = lax.shift_left(bu, jnp.uint32(4 - shift))
                idx = lax.bitcast_convert_type(
                    t & jnp.uint32((nbins - 1) * _L), jnp.int32) | lane
                m = (bu & jnp.uint32(maskc)) == pv
                plsc.addupdate_scatter(hist, [idx], ones, mask=m)

        last = _SLABS_PER_W - 1
        pltpu.make_async_copy(slab_src(0), buf0, sem0).start()
        pltpu.make_async_copy(slab_src(1), buf1, sem1).start()

        def pair(p, _):
            l0 = 2 * p
            pltpu.make_async_copy(slab_src(l0), buf0, sem0).wait()
            process(buf0)
            pltpu.make_async_copy(
                slab_src(jnp.minimum(l0 + 2, last)), buf0, sem0).start()
            pltpu.make_async_copy(slab_src(l0 + 1), buf1, sem1).wait()
            process(buf1)
            pltpu.make_async_copy(
                slab_src(jnp.minimum(l0 + 3, last)), buf1, sem1).start()
            return 0

        lax.fori_loop(0, _SLABS_PER_W // 2, pair, 0)
        pltpu.make_async_copy(slab_src(last), buf0, sem0).wait()
        pltpu.make_async_copy(slab_src(last), buf1, sem1).wait()

        pltpu.sync_copy(hist, out_hbm.at[wid])

    return hist_kernel


_hist_a = _make_hist_kernel(20, 4096, 0)
_hist_b = _make_hist_kernel(8, 4096, 0xFFF00000)
_hist_c = _make_hist_kernel(0, 256, 0xFFFFFF00)


def _make_hist16_kernel(top):
    nbins = 65536
    mesh = plsc.VectorSubcoreMesh(core_axis_name="c", subcore_axis_name="s")

    @functools.partial(
        pl.kernel,
        mesh=mesh,
        compiler_params=pltpu.CompilerParams(needs_layout_passes=False),
        out_type=jax.ShapeDtypeStruct((_NW, nbins), jnp.int32),
        scratch_types=[
            pltpu.VMEM((_SLAB_R, _SLAB_C), jnp.float32),
            pltpu.VMEM((_SLAB_R, _SLAB_C), jnp.float32),
            pltpu.VMEM((nbins,), jnp.int32),
            pltpu.VMEM((_L,), jnp.uint32),
            pltpu.SemaphoreType.DMA,
            pltpu.SemaphoreType.DMA,
        ],
    )
    def hist_kernel(scores_hbm, prefix_hbm, out_hbm, buf0, buf1, hist,
                    pref_v, sem0, sem1):
        wid = lax.axis_index("s") * _NC + lax.axis_index("c")

        def slab_src(l):
            g = wid * _SLABS_PER_W + l
            r0 = (g // _SLABS_PER_ROWBAND) * _SLAB_R
            c0 = (g % _SLABS_PER_ROWBAND) * _SLAB_C
            return scores_hbm.at[pl.ds(r0, _SLAB_R), pl.ds(c0, _SLAB_C)]

        pltpu.make_async_copy(slab_src(0), buf0, sem0).start()
        pltpu.make_async_copy(slab_src(1), buf1, sem1).start()

        zeros = jnp.zeros((_L,), jnp.int32)

        @plsc.parallel_loop(0, nbins // _L, unroll=8)
        def _zero(i):
            hist[pl.ds(i * _L, _L)] = zeros

        pltpu.sync_copy(prefix_hbm, pref_v)
        pv = pref_v[...]

        def process(buf):
            @plsc.parallel_loop(0, _SLAB_ELEMS // _L, unroll=16)
            def _body(j):
                r = lax.shift_right_logical(j, 7)
                c = (j & (_SLAB_C // _L - 1)) * _L
                v = buf[r, pl.ds(c, _L)]
                bu = lax.bitcast_convert_type(v, jnp.uint32)
                if top:
                    bucket = lax.shift_right_logical(bu, jnp.uint32(16))
                    m = (bu & jnp.uint32(0)) == pv
                else:
                    bucket = bu & jnp.uint32(0xFFFF)
                    m = (bu & jnp.uint32(0xFFFF0000)) == pv
                idx = lax.bitcast_convert_type(bucket, jnp.int32)
                cnt, last = plsc.scan_count(idx, mask=m)
                plsc.addupdate_scatter(hist, [idx], cnt, mask=last)

        last_slab = _SLABS_PER_W - 1

        def pair(p, _):
            l0 = 2 * p
            pltpu.make_async_copy(slab_src(l0), buf0, sem0).wait()
            process(buf0)
            pltpu.make_async_copy(
                slab_src(jnp.minimum(l0 + 2, last_slab)), buf0, sem0).start()
            pltpu.make_async_copy(slab_src(l0 + 1), buf1, sem1).wait()
            process(buf1)
            pltpu.make_async_copy(
                slab_src(jnp.minimum(l0 + 3, last_slab)), buf1, sem1).start()
            return 0

        lax.fori_loop(0, _SLABS_PER_W // 2, pair, 0)
        pltpu.make_async_copy(slab_src(last_slab), buf0, sem0).wait()
        pltpu.make_async_copy(slab_src(last_slab), buf1, sem1).wait()

        pltpu.sync_copy(hist, out_hbm.at[wid])

    return hist_kernel


_hist16_hi = _make_hist16_kernel(True)
_hist16_lo = _make_hist16_kernel(False)


def _reduce(hist_flat, nbins):
    return hist_flat.reshape(_NW, nbins, _L).sum(axis=(0, 2))


def _pick(h, rank):
    c = jnp.cumsum(h)
    b = jnp.argmax(c >= rank)
    within = rank - (c[b] - h[b])
    return b, within


def _apply_body(thr_ref, w_ref, s_ref, o_ref):
    thr = thr_ref[0, 0]
    o_ref[...] = jnp.where(s_ref[...] < thr, jnp.float32(0.0), w_ref[...])


_apply = pl.pallas_call(
    _apply_body,
    grid=(16,),
    in_specs=[
        pl.BlockSpec(memory_space=pltpu.SMEM),
        pl.BlockSpec((256, 4096), lambda i: (i, 0)),
        pl.BlockSpec((256, 4096), lambda i: (i, 0)),
    ],
    out_specs=pl.BlockSpec((256, 4096), lambda i: (i, 0)),
    out_shape=jax.ShapeDtypeStruct((4096, 4096), jnp.float32),
)


def kernel(weight, scores):
    n = scores.size
    k = jnp.int32(int(1 + round(0.9 * (n - 1))))

    h_raw = _hist16_hi(scores, jnp.zeros((_L,), jnp.uint32)).sum(axis=0)
    h_key = jnp.concatenate([h_raw[32768:][::-1], h_raw[:32768]])
    b1, r1 = _pick(h_key, k)
    neg = b1 < 32768
    r1raw = jnp.where(neg, 65535 - b1, b1 - 32768)
    p1 = r1raw.astype(jnp.uint32) << 16

    h2 = _hist16_lo(scores, jnp.broadcast_to(p1, (_L,))).sum(axis=0)
    b2, _ = _pick(jnp.where(neg, h2[::-1], h2), r1)
    lowraw = jnp.where(neg, 65535 - b2, b2)

    bits = p1 | lowraw.astype(jnp.uint32)
    thr = lax.bitcast_convert_type(bits, jnp.float32)

    return _apply(thr.reshape(1, 1), weight, scores)


# speedup vs baseline: 147.1104x; 1.0026x over previous
import functools

import jax
import jax.numpy as jnp
from jax import lax
from jax.experimental import pallas as pl
from jax.experimental.pallas import tpu as pltpu
from jax.experimental.pallas import tpu_sc as plsc

_N = 4096 * 4096
_NC = 2
_NS = 16
_NW = _NC * _NS
_L = 16
_PER_W = _N // _NW
_SLAB_R = 8
_SLAB_C = 2048
_SLAB_ELEMS = _SLAB_R * _SLAB_C
_SLABS_PER_W = _PER_W // _SLAB_ELEMS
_SLABS_PER_ROWBAND = 4096 // _SLAB_C


def _make_hist_kernel(shift, nbins, maskc):
    hist_words = nbins * _L
    mesh = plsc.VectorSubcoreMesh(core_axis_name="c", subcore_axis_name="s")

    @functools.partial(
        pl.kernel,
        mesh=mesh,
        compiler_params=pltpu.CompilerParams(needs_layout_passes=False),
        out_type=jax.ShapeDtypeStruct((_NW, hist_words), jnp.int32),
        scratch_types=[
            pltpu.VMEM((_SLAB_R, _SLAB_C), jnp.float32),
            pltpu.VMEM((_SLAB_R, _SLAB_C), jnp.float32),
            pltpu.VMEM((hist_words,), jnp.int32),
            pltpu.VMEM((_L,), jnp.uint32),
            pltpu.SemaphoreType.DMA,
            pltpu.SemaphoreType.DMA,
        ],
    )
    def hist_kernel(scores_hbm, prefix_hbm, out_hbm, buf0, buf1, hist,
                    pref_v, sem0, sem1):
        wid = lax.axis_index("s") * _NC + lax.axis_index("c")

        zeros = jnp.zeros((_L,), jnp.int32)

        @plsc.parallel_loop(0, nbins, unroll=8)
        def _zero(i):
            hist[pl.ds(i * _L, _L)] = zeros

        pltpu.sync_copy(prefix_hbm, pref_v)
        pv = pref_v[...]
        lane = lax.iota(jnp.int32, _L)
        ones = jnp.ones((_L,), jnp.int32)

        def slab_src(l):
            g = wid * _SLABS_PER_W + l
            r0 = (g // _SLABS_PER_ROWBAND) * _SLAB_R
            c0 = (g % _SLABS_PER_ROWBAND) * _SLAB_C
            return scores_hbm.at[pl.ds(r0, _SLAB_R), pl.ds(c0, _SLAB_C)]

        def process(buf):
            @plsc.parallel_loop(0, _SLAB_ELEMS // _L, unroll=8)
            def _body(j):
                r = lax.shift_right_logical(j, 7)
                c = (j & (_SLAB_C // _L - 1)) * _L
                v = buf[r, pl.ds(c, _L)]
                bu = lax.bitcast_convert_type(v, jnp.uint32)
                if shift >= 4:
                    t = lax.shift_right_logical(bu, jnp.uint32(shift - 4))
                else:
                    t = lax.shift_left(bu, jnp.uint32(4 - shift))
                idx = lax.bitcast_convert_type(
                    t & jnp.uint32((nbins - 1) * _L), jnp.int32) | lane
                m = (bu & jnp.uint32(maskc)) == pv
                plsc.addupdate_scatter(hist, [idx], ones, mask=m)

        last = _SLABS_PER_W - 1
        pltpu.make_async_copy(slab_src(0), buf0, sem0).start()
        pltpu.make_async_copy(slab_src(1), buf1, sem1).start()

        def pair(p, _):
            l0 = 2 * p
            pltpu.make_async_copy(slab_src(l0), buf0, sem0).wait()
            process(buf0)
            pltpu.make_async_copy(
                slab_src(jnp.minimum(l0 + 2, last)), buf0, sem0).start()
            pltpu.make_async_copy(slab_src(l0 + 1), buf1, sem1).wait()
            process(buf1)
            pltpu.make_async_copy(
                slab_src(jnp.minimum(l0 + 3, last)), buf1, sem1).start()
            return 0

        lax.fori_loop(0, _SLABS_PER_W // 2, pair, 0)
        pltpu.make_async_copy(slab_src(last), buf0, sem0).wait()
        pltpu.make_async_copy(slab_src(last), buf1, sem1).wait()

        pltpu.sync_copy(hist, out_hbm.at[wid])

    return hist_kernel


_hist_a = _make_hist_kernel(20, 4096, 0)
_hist_b = _make_hist_kernel(8, 4096, 0xFFF00000)
_hist_c = _make_hist_kernel(0, 256, 0xFFFFFF00)


def _make_hist16_kernel(top):
    nbins = 65536
    mesh = plsc.VectorSubcoreMesh(core_axis_name="c", subcore_axis_name="s")

    @functools.partial(
        pl.kernel,
        mesh=mesh,
        compiler_params=pltpu.CompilerParams(needs_layout_passes=False),
        out_type=jax.ShapeDtypeStruct((_NW, nbins), jnp.int32),
        scratch_types=[
            pltpu.VMEM((_SLAB_R, _SLAB_C), jnp.float32),
            pltpu.VMEM((_SLAB_R, _SLAB_C), jnp.float32),
            pltpu.VMEM((nbins,), jnp.int32),
            pltpu.VMEM((_L,), jnp.uint32),
            pltpu.SemaphoreType.DMA,
            pltpu.SemaphoreType.DMA,
        ],
    )
    def hist_kernel(scores_hbm, prefix_hbm, out_hbm, buf0, buf1, hist,
                    pref_v, sem0, sem1):
        wid = lax.axis_index("s") * _NC + lax.axis_index("c")

        def slab_src(l):
            g = wid * _SLABS_PER_W + l
            r0 = (g // _SLABS_PER_ROWBAND) * _SLAB_R
            c0 = (g % _SLABS_PER_ROWBAND) * _SLAB_C
            return scores_hbm.at[pl.ds(r0, _SLAB_R), pl.ds(c0, _SLAB_C)]

        pltpu.make_async_copy(slab_src(0), buf0, sem0).start()
        pltpu.make_async_copy(slab_src(1), buf1, sem1).start()

        zeros = jnp.zeros((_L,), jnp.int32)

        @plsc.parallel_loop(0, nbins // _L, unroll=8)
        def _zero(i):
            hist[pl.ds(i * _L, _L)] = zeros

        pltpu.sync_copy(prefix_hbm, pref_v)
        pv = pref_v[...]
        ones = jnp.ones((_L,), jnp.int32)

        def process(buf):
            @plsc.parallel_loop(0, _SLAB_ELEMS // _L, unroll=16)
            def _body(j):
                r = lax.shift_right_logical(j, 7)
                c = (j & (_SLAB_C // _L - 1)) * _L
                v = buf[r, pl.ds(c, _L)]
                bu = lax.bitcast_convert_type(v, jnp.uint32)
                if top:
                    bucket = lax.shift_right_logical(bu, jnp.uint32(16))
                    m = (bu & jnp.uint32(0)) == pv
                else:
                    bucket = bu & jnp.uint32(0xFFFF)
                    m = (bu & jnp.uint32(0xFFFF0000)) == pv
                idx = lax.bitcast_convert_type(bucket, jnp.int32)
                plsc.addupdate_scatter(hist, [idx], ones, mask=m)

        last_slab = _SLABS_PER_W - 1

        def pair(p, _):
            l0 = 2 * p
            pltpu.make_async_copy(slab_src(l0), buf0, sem0).wait()
            process(buf0)
            pltpu.make_async_copy(
                slab_src(jnp.minimum(l0 + 2, last_slab)), buf0, sem0).start()
            pltpu.make_async_copy(slab_src(l0 + 1), buf1, sem1).wait()
            process(buf1)
            pltpu.make_async_copy(
                slab_src(jnp.minimum(l0 + 3, last_slab)), buf1, sem1).start()
            return 0

        lax.fori_loop(0, _SLABS_PER_W // 2, pair, 0)
        pltpu.make_async_copy(slab_src(last_slab), buf0, sem0).wait()
        pltpu.make_async_copy(slab_src(last_slab), buf1, sem1).wait()

        pltpu.sync_copy(hist, out_hbm.at[wid])

    return hist_kernel


_hist16_hi = _make_hist16_kernel(True)
_hist16_lo = _make_hist16_kernel(False)


def _reduce(hist_flat, nbins):
    return hist_flat.reshape(_NW, nbins, _L).sum(axis=(0, 2))


def _pick(h, rank):
    c = jnp.cumsum(h)
    b = jnp.argmax(c >= rank)
    within = rank - (c[b] - h[b])
    return b, within


def _apply_body(thr_ref, w_ref, s_ref, o_ref):
    thr = thr_ref[0, 0]
    o_ref[...] = jnp.where(s_ref[...] < thr, jnp.float32(0.0), w_ref[...])


_apply = pl.pallas_call(
    _apply_body,
    grid=(16,),
    in_specs=[
        pl.BlockSpec(memory_space=pltpu.SMEM),
        pl.BlockSpec((256, 4096), lambda i: (i, 0)),
        pl.BlockSpec((256, 4096), lambda i: (i, 0)),
    ],
    out_specs=pl.BlockSpec((256, 4096), lambda i: (i, 0)),
    out_shape=jax.ShapeDtypeStruct((4096, 4096), jnp.float32),
)


def kernel(weight, scores):
    n = scores.size
    k = jnp.int32(int(1 + round(0.9 * (n - 1))))

    h_raw = _hist16_hi(scores, jnp.zeros((_L,), jnp.uint32)).sum(axis=0)
    h_key = jnp.concatenate([h_raw[32768:][::-1], h_raw[:32768]])
    b1, r1 = _pick(h_key, k)
    neg = b1 < 32768
    r1raw = jnp.where(neg, 65535 - b1, b1 - 32768)
    p1 = r1raw.astype(jnp.uint32) << 16

    h2 = _hist16_lo(scores, jnp.broadcast_to(p1, (_L,))).sum(axis=0)
    b2, _ = _pick(jnp.where(neg, h2[::-1], h2), r1)
    lowraw = jnp.where(neg, 65535 - b2, b2)

    bits = p1 | lowraw.astype(jnp.uint32)
    thr = lax.bitcast_convert_type(bits, jnp.float32)

    return _apply(thr.reshape(1, 1), weight, scores)
